# baseline (speedup 1.0000x reference)
import functools

import jax
import jax.numpy as jnp
from jax import lax
from jax.experimental import pallas as pl
from jax.experimental.pallas import tpu as pltpu
from jax.experimental.pallas import tpu_sc as plsc

_NC, _NS, _VEC = 2, 16, 16
_NW = _NC * _NS
_CH = 128
_SCH = 3 * _CH


def _comb_body(pos_ref, tab_ref, out_ref):
    out_ref[0] = pos_ref[...] + tab_ref[0, :][None, :]
    out_ref[1] = pos_ref[...] + tab_ref[1, :][None, :]


def _build_comb(pos, mask_table, t, c):
    return pl.pallas_call(
        _comb_body,
        in_specs=[
            pl.BlockSpec((t, c), lambda: (0, 0)),
            pl.BlockSpec((2, c), lambda: (0, 0)),
        ],
        out_specs=pl.BlockSpec((2, t, c), lambda: (0, 0, 0)),
        out_shape=jax.ShapeDtypeStruct((2, t, c), jnp.float32),
    )(pos, mask_table)


def _make_sc_gather(tok, t, c):
    per_w = tok // _NW
    nch = per_w // _CH
    nsc = -(-nch // 3)
    tail_ch = nch - 3 * (nsc - 1)
    mesh = plsc.VectorSubcoreMesh(
        core_axis_name="c", subcore_axis_name="s",
        num_cores=_NC, num_subcores=_NS,
    )

    @functools.partial(
        pl.kernel,
        out_type=jax.ShapeDtypeStruct((tok, c), jnp.float32),
        mesh=mesh,
        scratch_types=[
            pltpu.VMEM_SHARED((2 * t, c), jnp.float32),
            pltpu.VMEM((per_w,), jnp.int32),
            pltpu.VMEM((nch, _CH), jnp.int32),
            pltpu.VMEM((2, _SCH, c), jnp.float32),
            pltpu.SemaphoreType.DMA,
            pltpu.SemaphoreType.DMA,
        ],
    )
    def sc_gather(comb_hbm, mask_hbm, out_hbm,
                  comb_sh, mask_v, idx_v, bufs, sem0, sem1):
        sid = lax.axis_index("s")
        wid = sid * _NC + lax.axis_index("c")
        base = wid * per_w

        @pl.when(sid == 0)
        def _():
            pltpu.sync_copy(comb_hbm, comb_sh)

        pltpu.sync_copy(mask_hbm.at[pl.ds(base, per_w)], mask_v)

        lanes = lax.iota(jnp.int32, _VEC)

        def idx_row(j, _):
            def idx_vec(v, _):
                p = j * _CH + v * _VEC
                m = mask_v[pl.ds(p, _VEC)]
                tpos = lax.rem(base + p + lanes, t)
                idx_v[j, pl.ds(v * _VEC, _VEC)] = m * t + tpos
                return 0
            return lax.fori_loop(0, _CH // _VEC, idx_vec, 0)

        lax.fori_loop(0, 6, idx_row, 0)
        plsc.subcore_barrier()

        b0 = bufs.at[0]
        b1 = bufs.at[1]

        def fire(s, buf, sem, k):
            for i in range(k):
                pltpu.async_copy(comb_sh.at[idx_v.at[3 * s + i]],
                                 buf.at[pl.ds(i * _CH, _CH)], sem)

        def drain(buf, sem, k):
            pltpu.make_async_copy(out_hbm.at[pl.ds(0, k * _CH)],
                                  buf.at[pl.ds(0, k * _CH)], sem).wait()

        def scatter(s, buf, k):
            pltpu.sync_copy(buf.at[pl.ds(0, k * _CH)],
                            out_hbm.at[pl.ds(base + s * _SCH, k * _CH)])

        fire(0, b0, sem0, 3)
        fire(1, b1, sem1, 3)

        lax.fori_loop(6, nch, idx_row, 0)

        def pair(g, _):
            s0 = 2 * g
            s1 = s0 + 1
            drain(b0, sem0, 3)
            scatter(s0, b0, 3)

            @pl.when(s0 + 2 < nsc - 1)
            def _():
                fire(s0 + 2, b0, sem0, 3)

            @pl.when(s0 + 2 == nsc - 1)
            def _():
                fire(nsc - 1, b0, sem0, tail_ch)

            drain(b1, sem1, 3)
            scatter(s1, b1, 3)

            @pl.when(s1 + 2 < nsc - 1)
            def _():
                fire(s1 + 2, b1, sem1, 3)
            return 0

        lax.fori_loop(0, nsc // 2, pair, 0)

        drain(b0, sem0, tail_ch)
        scatter(nsc - 1, b0, tail_ch)

    return sc_gather


def kernel(x, mask, pos_emb, mask_table):
    b, t, c = x.shape
    tok = b * t
    pos = pos_emb[0, :t, :]
    m_flat = mask.astype(jnp.int32).reshape(tok)
    comb = _build_comb(pos, mask_table, t, c).reshape(2 * t, c)
    out = _make_sc_gather(tok, t, c)(comb, m_flat)
    return out.reshape(b, t, c)
